# baseline (speedup 1.0000x reference)
import functools

import jax
import jax.numpy as jnp
from jax import lax
from jax.experimental import pallas as pl
from jax.experimental.pallas import tpu as pltpu
from jax.experimental.pallas import tpu_sc as plsc

B, H = 4096, 50
DL, DT = 64, 32
D = DL + DT
NLOC, NTIME = 4, 3
NC, NS = 2, 16
NW = NC * NS
C = 128
NBLK = B // C
NB = 2


def _body(xw, tw, wl0, wl1, wl2, wl3, wt0, wt1, wt2, out,
          xi0, xi1, ti0, ti1, r00, r01, r10, r11,
          s00, s01, s10, s11, o0, o1,
          gsem0, gsem1, osem0, osem1, isem0, isem1):
    wid = lax.axis_index("s") * NC + lax.axis_index("c")

    xid = [xi0, xi1]
    tid = [ti0, ti1]
    rl = [[r00, r01], [r10, r11]]
    rt = [[s00, s01], [s10, s11]]
    ostage = [o0, o1]
    gsem = [gsem0, gsem1]
    osem = [osem0, osem1]
    isem = [isem0, isem1]

    def fire_idx(h, b):
        pltpu.async_copy(xw.at[h, wid], xid[b], isem[b])
        pltpu.async_copy(tw.at[h, wid], tid[b], isem[b])

    def wait_idx(b):
        pltpu.make_async_copy(xw.at[0, 0], xid[b], isem[b]).wait()
        pltpu.make_async_copy(tw.at[0, 0], tid[b], isem[b]).wait()

    def fire_wave1(b):
        wait_idx(b)
        pltpu.async_copy(wl0.at[xid[b].at[0]], rl[b][0], gsem[b])
        pltpu.async_copy(wl1.at[xid[b].at[1]], rl[b][1], gsem[b])
        pltpu.async_copy(wt0.at[tid[b].at[0]], rt[b][0], gsem[b])
        pltpu.async_copy(wt1.at[tid[b].at[1]], rt[b][1], gsem[b])

    def fire_wave2(b):
        pltpu.async_copy(wl2.at[xid[b].at[2]], rl[b][0], gsem[b], add=True)
        pltpu.async_copy(wl3.at[xid[b].at[3]], rl[b][1], gsem[b], add=True)
        pltpu.async_copy(wt2.at[tid[b].at[2]], rt[b][0], gsem[b], add=True)

    def drain_wave1(b):
        for r in rl[b]:
            pltpu.make_async_copy(wl0.at[pl.ds(0, C)], r, gsem[b]).wait()
        for s in rt[b]:
            pltpu.make_async_copy(wt0.at[pl.ds(0, C)], s, gsem[b]).wait()

    def drain_wave2(b):
        pltpu.make_async_copy(wl0.at[pl.ds(0, C)], rl[b][0], gsem[b]).wait()
        pltpu.make_async_copy(wl0.at[pl.ds(0, C)], rl[b][1], gsem[b]).wait()
        pltpu.make_async_copy(wt0.at[pl.ds(0, C)], rt[b][0], gsem[b]).wait()

    def drain_out(b):
        pltpu.make_async_copy(ostage[b].at[:, :, pl.ds(0, C)],
                              out.at[0, :, 0], osem[b]).wait()

    lanes = lax.iota(jnp.int32, 16)
    dvs = [lanes + 16 * j for j in range(D // 16)]
    trs = [dv >> 3 for dv in dvs]
    srs = [dv & 7 for dv in dvs]

    def compute(b):
        r0, r1 = rl[b]
        s0, s1 = rt[b]
        ob = ostage[b]

        @plsc.parallel_loop(0, C, unroll=2)
        def add_row(i):
            pcol = lanes * 0 + i
            for j in range(DL // 16):
                sl = pl.ds(j * 16, 16)
                plsc.store_scatter(ob, [trs[j], srs[j], pcol],
                                   r0[i, sl] + r1[i, sl])
            for j in range(DT // 16):
                sl = pl.ds(j * 16, 16)
                jj = DL // 16 + j
                plsc.store_scatter(ob, [trs[jj], srs[jj], pcol],
                                   s0[i, sl] + s1[i, sl])

    for b in range(NB):
        fire_idx(b, b)
    fire_wave1(0)
    drain_wave1(0)
    fire_wave2(0)
    fire_wave1(1)

    def step(i, _):
        for b in range(NB):
            h = i * NB + b
            drain_wave2(b)

            @pl.when(h + NB < H)
            def _i():
                fire_idx(h + NB, b)

            @pl.when(i > 0)
            def _w():
                drain_out(b)

            compute(b)
            pltpu.async_copy(ostage[b].at[:, :, pl.ds(0, C)],
                             out.at[h, :, wid], osem[b])

            @pl.when(h + 1 < H)
            def _n():
                drain_wave1(1 - b)
                fire_wave2(1 - b)

            @pl.when(h + NB < H)
            def _f():
                fire_wave1(b)
        return _

    lax.fori_loop(0, H // NB, step, None)
    for b in range(NB):
        drain_out(b)


@jax.jit
def _emb(xw, tw, wl0, wl1, wl2, wl3, wt0, wt1, wt2):
    mesh = plsc.VectorSubcoreMesh(core_axis_name="c", subcore_axis_name="s")
    scratch = (
        [pltpu.VMEM((NLOC, C), jnp.int32) for _ in range(NB)]
        + [pltpu.VMEM((NTIME, C), jnp.int32) for _ in range(NB)]
        + [pltpu.VMEM((C, DL), jnp.float32) for _ in range(NB * 2)]
        + [pltpu.VMEM((C, DT), jnp.float32) for _ in range(NB * 2)]
        + [pltpu.VMEM((D // 8, 8, C + 1), jnp.float32) for _ in range(NB)]
        + [pltpu.SemaphoreType.DMA for _ in range(3 * NB)]
    )
    return pl.kernel(
        _body,
        out_type=jax.ShapeDtypeStruct((H, D // 8, NBLK, 8, C), jnp.float32),
        mesh=mesh,
        scratch_types=scratch,
        compiler_params=pltpu.CompilerParams(use_tc_tiling_on_sc=False,
                                             needs_layout_passes=False),
    )(xw, tw, wl0, wl1, wl2, wl3, wt0, wt1, wt2)


def kernel(x, t, W_loc0, W_loc1, W_loc2, W_loc3, W_time0, W_time1, W_time2):
    xw = (x.astype(jnp.int32).transpose(1, 0, 2)
          .reshape(H, NBLK, C, NLOC).transpose(0, 1, 3, 2))
    tw = (t.astype(jnp.int32).transpose(1, 0, 2)
          .reshape(H, NBLK, C, NTIME).transpose(0, 1, 3, 2))
    out5 = _emb(xw, tw, W_loc0, W_loc1, W_loc2, W_loc3,
                W_time0, W_time1, W_time2)
    return out5.transpose(2, 4, 0, 1, 3).reshape(B, H, D)


# speedup vs baseline: 1.1095x; 1.1095x over previous
import functools

import jax
import jax.numpy as jnp
from jax import lax
from jax.experimental import pallas as pl
from jax.experimental.pallas import tpu as pltpu
from jax.experimental.pallas import tpu_sc as plsc

B, H = 4096, 50
DL, DT = 64, 32
D = DL + DT
NLOC, NTIME = 4, 3
NC, NS = 2, 16
NW = NC * NS
C = 128
NBLK = B // C
NB = 2


def _body(xw, tw, wl0, wl1, wl2, wl3, wt0, wt1, wt2, out,
          xi0, xi1, ti0, ti1, r00, r01, r02, r03, r10, r11, r12, r13,
          s00, s01, s02, s10, s11, s12, o0, o1,
          gsem0, gsem1, osem0, osem1, isem0, isem1):
    wid = lax.axis_index("s") * NC + lax.axis_index("c")

    xid = [xi0, xi1]
    tid = [ti0, ti1]
    rl = [[r00, r01, r02, r03], [r10, r11, r12, r13]]
    rt = [[s00, s01, s02], [s10, s11, s12]]
    ostage = [o0, o1]
    gsem = [gsem0, gsem1]
    osem = [osem0, osem1]
    isem = [isem0, isem1]
    wls = [wl0, wl1, wl2, wl3]
    wts = [wt0, wt1, wt2]

    def fire_idx(h, b):
        pltpu.async_copy(xw.at[h, wid], xid[b], isem[b])
        pltpu.async_copy(tw.at[h, wid], tid[b], isem[b])

    def fire_gathers(b):
        pltpu.make_async_copy(xw.at[0, 0], xid[b], isem[b]).wait()
        pltpu.make_async_copy(tw.at[0, 0], tid[b], isem[b]).wait()
        for l in range(NLOC):
            pltpu.async_copy(wls[l].at[xid[b].at[l]], rl[b][l], gsem[b])
        for l in range(NTIME):
            pltpu.async_copy(wts[l].at[tid[b].at[l]], rt[b][l], gsem[b])

    def drain_gathers(b):
        for l in range(NLOC):
            pltpu.make_async_copy(wls[l].at[pl.ds(0, C)], rl[b][l],
                                  gsem[b]).wait()
        for l in range(NTIME):
            pltpu.make_async_copy(wts[l].at[pl.ds(0, C)], rt[b][l],
                                  gsem[b]).wait()

    def drain_out(b):
        pltpu.make_async_copy(ostage[b].at[:, :, pl.ds(0, C)],
                              out.at[0, :, 0], osem[b]).wait()

    lanes = lax.iota(jnp.int32, 16)
    dvs = [lanes + 16 * j for j in range(D // 16)]
    trs = [dv >> 3 for dv in dvs]
    srs = [dv & 7 for dv in dvs]

    def compute(b):
        r0, r1, r2, r3 = rl[b]
        s0, s1, s2 = rt[b]
        ob = ostage[b]

        @plsc.parallel_loop(0, C, unroll=4)
        def add_row(i):
            pcol = lanes * 0 + i
            for j in range(DL // 16):
                sl = pl.ds(j * 16, 16)
                v = (r0[i, sl] + r1[i, sl]) + (r2[i, sl] + r3[i, sl])
                plsc.store_scatter(ob, [trs[j], srs[j], pcol], v)
            for j in range(DT // 16):
                sl = pl.ds(j * 16, 16)
                v = (s0[i, sl] + s1[i, sl]) + s2[i, sl]
                jj = DL // 16 + j
                plsc.store_scatter(ob, [trs[jj], srs[jj], pcol], v)

    for b in range(NB):
        fire_idx(b, b)
    for b in range(NB):
        fire_gathers(b)

    def step(i, _):
        for b in range(NB):
            h = i * NB + b
            drain_gathers(b)

            @pl.when(h + NB < H)
            def _i():
                fire_idx(h + NB, b)

            @pl.when(i > 0)
            def _w():
                drain_out(b)

            compute(b)
            pltpu.async_copy(ostage[b].at[:, :, pl.ds(0, C)],
                             out.at[h, :, wid], osem[b])

            @pl.when(h + NB < H)
            def _f():
                fire_gathers(b)
        return _

    lax.fori_loop(0, H // NB, step, None)
    for b in range(NB):
        drain_out(b)


@jax.jit
def _emb(xw, tw, wl0, wl1, wl2, wl3, wt0, wt1, wt2):
    mesh = plsc.VectorSubcoreMesh(core_axis_name="c", subcore_axis_name="s")
    scratch = (
        [pltpu.VMEM((NLOC, C), jnp.int32) for _ in range(NB)]
        + [pltpu.VMEM((NTIME, C), jnp.int32) for _ in range(NB)]
        + [pltpu.VMEM((C, DL), jnp.float32) for _ in range(NB * NLOC)]
        + [pltpu.VMEM((C, DT), jnp.float32) for _ in range(NB * NTIME)]
        + [pltpu.VMEM((D // 8, 8, C + 1), jnp.float32) for _ in range(NB)]
        + [pltpu.SemaphoreType.DMA for _ in range(3 * NB)]
    )
    return pl.kernel(
        _body,
        out_type=jax.ShapeDtypeStruct((H, D // 8, NBLK, 8, C), jnp.float32),
        mesh=mesh,
        scratch_types=scratch,
        compiler_params=pltpu.CompilerParams(use_tc_tiling_on_sc=False,
                                             needs_layout_passes=False),
    )(xw, tw, wl0, wl1, wl2, wl3, wt0, wt1, wt2)


def kernel(x, t, W_loc0, W_loc1, W_loc2, W_loc3, W_time0, W_time1, W_time2):
    xw = (x.astype(jnp.int32).transpose(1, 0, 2)
          .reshape(H, NBLK, C, NLOC).transpose(0, 1, 3, 2))
    tw = (t.astype(jnp.int32).transpose(1, 0, 2)
          .reshape(H, NBLK, C, NTIME).transpose(0, 1, 3, 2))
    out5 = _emb(xw, tw, W_loc0, W_loc1, W_loc2, W_loc3,
                W_time0, W_time1, W_time2)
    return out5.transpose(2, 4, 0, 1, 3).reshape(B, H, D)
